# Initial kernel scaffold; baseline (speedup 1.0000x reference)
import jax
import jax.numpy as jnp
from jax.experimental import pallas as pl


def kernel(cond_table, cat_table, cond_ids, cat_ids, cat_start):
    raise NotImplementedError("write your pallas kernel here")


# speedup vs baseline: 2.0448x; 2.0448x over previous
import jax
import jax.numpy as jnp
from jax import lax
from jax.experimental import pallas as pl
from jax.experimental.pallas import tpu as pltpu
from jax.experimental.pallas import tpu_sc as plsc

N_COND = 26
N_CAT = 38461
EMBED = 32
B = 16384

N_TOT = B * N_COND
NW = 32
CHUNK = N_TOT // NW
SUB = 128
NSUB = CHUNK // SUB
GROUPS = SUB // 16


def _body(cond_flat, cat_table, cond_ids, cat_ids, cat_start, out,
          cs_v, cf_v, ci_v, ca_v, idx2, rb):
    wid = lax.axis_index("s") * 2 + lax.axis_index("c")
    base = wid * CHUNK

    pltpu.sync_copy(cat_start, cs_v)
    pltpu.sync_copy(cond_flat, cf_v)
    pltpu.sync_copy(cond_ids.at[pl.ds(base, CHUNK)], ci_v)
    pltpu.sync_copy(cat_ids.at[pl.ds(base, CHUNK)], ca_v)

    def idx_pass(j, carry):
        for k in range(GROUPS):
            off = j * SUB + k * 16
            c16 = ci_v[pl.ds(off, 16)]
            a16 = ca_v[pl.ds(off, 16)]
            s16 = plsc.load_gather(cs_v, [c16])
            idx2[j, pl.ds(k * 16, 16)] = s16 + a16
        return carry

    lax.fori_loop(0, NSUB, idx_pass, 0, unroll=False)

    iota = lax.iota(jnp.int32, 16)

    def main_pass(j, carry):
        pltpu.sync_copy(cat_table.at[idx2.at[j]], rb)
        for k in range(GROUPS):
            off = j * SUB + k * 16
            cid = ci_v[pl.ds(off, 16)]
            cb = cid * EMBED
            rows16 = iota + (k * 16)
            for c in range(EMBED):
                vec = plsc.load_gather(cf_v, [cb + c])
                col16 = jnp.full((16,), c, jnp.int32)
                plsc.addupdate_scatter(rb, [rows16, col16], vec)
        pltpu.sync_copy(rb, out.at[pl.ds(base + j * SUB, SUB)])
        return carry

    lax.fori_loop(0, NSUB, main_pass, 0, unroll=False)


def kernel(cond_table, cat_table, cond_ids, cat_ids, cat_start):
    cond_flat = cond_table.reshape(-1)
    ci = cond_ids.reshape(-1)
    ca = cat_ids.reshape(-1)
    cs = jnp.pad(cat_start, (0, 32 - cat_start.shape[0]))

    mesh = plsc.VectorSubcoreMesh(core_axis_name="c", subcore_axis_name="s")
    f = pl.kernel(
        _body,
        out_type=jax.ShapeDtypeStruct((N_TOT, EMBED), jnp.float32),
        mesh=mesh,
        compiler_params=pltpu.CompilerParams(needs_layout_passes=False,
                                             use_tc_tiling_on_sc=False),
        scratch_types=[
            pltpu.VMEM((32,), jnp.int32),
            pltpu.VMEM((EMBED * (N_COND + 1),), jnp.float32),
            pltpu.VMEM((CHUNK,), jnp.int32),
            pltpu.VMEM((CHUNK,), jnp.int32),
            pltpu.VMEM((NSUB, SUB), jnp.int32),
            pltpu.VMEM((SUB, EMBED), jnp.float32),
        ],
    )
    out = f(cond_flat, cat_table, ci, ca, cs)
    return out.reshape(B, N_COND, EMBED)


# speedup vs baseline: 2.2007x; 1.0762x over previous
import jax
import jax.numpy as jnp
from jax import lax
from jax.experimental import pallas as pl
from jax.experimental.pallas import tpu as pltpu
from jax.experimental.pallas import tpu_sc as plsc

N_COND = 26
N_CAT = 38461
EMBED = 32
B = 16384

N_TOT = B * N_COND
NW = 32
CHUNK = N_TOT // NW
SUB = 128
NSUB = CHUNK // SUB
GROUPS = SUB // 16
NBUF = 4
NROUND = NSUB // NBUF


def _body(cond_flat, cat_table, cond_ids, cat_ids, cat_start, out,
          cs_v, cf_v, ci_v, ca_v, idx2, rb0, rb1, rb2, rb3,
          g0, g1, g2, g3, o0, o1, o2, o3):
    rbs = [rb0, rb1, rb2, rb3]
    gsems = [g0, g1, g2, g3]
    osems = [o0, o1, o2, o3]

    wid = lax.axis_index("s") * 2 + lax.axis_index("c")
    base = wid * CHUNK

    pltpu.sync_copy(cat_start, cs_v)
    pltpu.sync_copy(cond_flat, cf_v)
    pltpu.sync_copy(cond_ids.at[pl.ds(base, CHUNK)], ci_v)
    pltpu.sync_copy(cat_ids.at[pl.ds(base, CHUNK)], ca_v)

    def idx_pass(j, carry):
        for k in range(GROUPS):
            off = j * SUB + k * 16
            c16 = ci_v[pl.ds(off, 16)]
            a16 = ca_v[pl.ds(off, 16)]
            s16 = plsc.load_gather(cs_v, [c16])
            idx2[j, pl.ds(k * 16, 16)] = s16 + a16
        return carry

    lax.fori_loop(0, NSUB, idx_pass, 0, unroll=False)

    iota = lax.iota(jnp.int32, 16)

    def fire_gather(j, s):
        pltpu.async_copy(cat_table.at[idx2.at[j]], rbs[s], gsems[s])

    def wait_gather(j, s):
        pltpu.make_async_copy(cat_table.at[idx2.at[j]], rbs[s],
                              gsems[s]).wait()

    def fire_out(j, s):
        pltpu.async_copy(rbs[s], out.at[pl.ds(base + j * SUB, SUB)],
                         osems[s])

    def wait_out(j, s):
        pltpu.make_async_copy(rbs[s], out.at[pl.ds(base + j * SUB, SUB)],
                              osems[s]).wait()

    def process(j, s):
        rb = rbs[s]

        def grp(k, carry):
            off = j * SUB + k * 16
            cid = ci_v[pl.ds(off, 16)]
            cb = cid * EMBED
            rows16 = iota + k * 16
            for c in range(EMBED):
                vec = plsc.load_gather(cf_v, [cb + c])
                col16 = jnp.full((16,), c, jnp.int32)
                plsc.addupdate_scatter(rb, [rows16, col16], vec)
            return carry

        lax.fori_loop(0, GROUPS, grp, 0, unroll=False)

    def step(j, s, prefetch):
        wait_gather(j, s)
        process(j, s)
        fire_out(j, s)
        if prefetch:
            p = (s - 1) % NBUF
            wait_out(j - 1, p)
            fire_gather(j - 1 + NBUF, p)

    for s in range(NBUF):
        fire_gather(s, s)
    step(0, 0, False)
    for s in range(1, NBUF):
        step(s, s, True)

    def round_body(r, carry):
        j0 = r * NBUF
        for s in range(NBUF):
            step(j0 + s, s, True)
        return carry

    lax.fori_loop(1, NROUND - 1, round_body, 0, unroll=False)

    j0 = (NROUND - 1) * NBUF
    step(j0, 0, True)
    for s in range(1, NBUF):
        step(j0 + s, s, False)
    for s in range(NBUF):
        wait_out(j0 + s, s)


def kernel(cond_table, cat_table, cond_ids, cat_ids, cat_start):
    cond_flat = cond_table.reshape(-1)
    ci = cond_ids.reshape(-1)
    ca = cat_ids.reshape(-1)
    cs = jnp.pad(cat_start, (0, 32 - cat_start.shape[0]))

    mesh = plsc.VectorSubcoreMesh(core_axis_name="c", subcore_axis_name="s")
    f = pl.kernel(
        _body,
        out_type=jax.ShapeDtypeStruct((N_TOT, EMBED), jnp.float32),
        mesh=mesh,
        compiler_params=pltpu.CompilerParams(needs_layout_passes=False,
                                             use_tc_tiling_on_sc=False),
        scratch_types=(
            [
                pltpu.VMEM((32,), jnp.int32),
                pltpu.VMEM((EMBED * (N_COND + 1),), jnp.float32),
                pltpu.VMEM((CHUNK,), jnp.int32),
                pltpu.VMEM((CHUNK,), jnp.int32),
                pltpu.VMEM((NSUB, SUB), jnp.int32),
            ]
            + [pltpu.VMEM((SUB, EMBED), jnp.float32) for _ in range(NBUF)]
            + [pltpu.SemaphoreType.DMA for _ in range(2 * NBUF)]
        ),
    )
    out = f(cond_flat, cat_table, ci, ca, cs)
    return out.reshape(B, N_COND, EMBED)


# speedup vs baseline: 2.9752x; 1.3519x over previous
import jax
import jax.numpy as jnp
from jax import lax
from jax.experimental import pallas as pl
from jax.experimental.pallas import tpu as pltpu
from jax.experimental.pallas import tpu_sc as plsc

N_COND = 26
N_CAT = 38461
EMBED = 32
B = 16384

N_TOT = B * N_COND
NW = 32
CHUNK = N_TOT // NW
SUB = 128
NSUB = CHUNK // SUB
GROUPS = SUB // 16
NBUF = 4
NROUND = NSUB // NBUF


def _body(cond_flat, cat_table, cond_ids, cat_ids, cat_start, out,
          cs_v, cf_v, ci_v, ca_v, idx2, cb_v, rb0, rb1, rb2, rb3,
          g0, g1, g2, g3, o0, o1, o2, o3):
    rbs = [rb0, rb1, rb2, rb3]
    gsems = [g0, g1, g2, g3]
    osems = [o0, o1, o2, o3]

    wid = lax.axis_index("s") * 2 + lax.axis_index("c")
    base = wid * CHUNK

    pltpu.sync_copy(cat_start, cs_v)
    pltpu.sync_copy(cond_flat, cf_v)
    pltpu.sync_copy(cond_ids.at[pl.ds(base, CHUNK)], ci_v)
    pltpu.sync_copy(cat_ids.at[pl.ds(base, CHUNK)], ca_v)

    def idx_pass(j, carry):
        for k in range(GROUPS):
            off = j * SUB + k * 16
            c16 = ci_v[pl.ds(off, 16)]
            a16 = ca_v[pl.ds(off, 16)]
            s16 = plsc.load_gather(cs_v, [c16])
            idx2[j, pl.ds(k * 16, 16)] = s16 + a16
            cb_v[pl.ds(off, 16)] = c16 * EMBED
        return carry

    lax.fori_loop(0, NSUB, idx_pass, 0, unroll=False)

    iota = lax.iota(jnp.int32, 16)

    def fire_gather(j, s):
        pltpu.async_copy(cat_table.at[idx2.at[j]], rbs[s], gsems[s])

    def wait_gather(j, s):
        pltpu.make_async_copy(cat_table.at[idx2.at[j]], rbs[s],
                              gsems[s]).wait()

    def fire_out(j, s):
        pltpu.async_copy(rbs[s], out.at[pl.ds(base + j * SUB, SUB)],
                         osems[s])

    def wait_out(j, s):
        pltpu.make_async_copy(rbs[s], out.at[pl.ds(base + j * SUB, SUB)],
                              osems[s]).wait()

    def process(j, s):
        rb = rbs[s]
        off = j * SUB

        def rows(r0, carry):
            for u in range(4):
                r = r0 * 4 + u
                cb16 = plsc.load_gather(cb_v, [jnp.full((16,), off, jnp.int32) + r])
                a0 = cb16 + iota
                c0 = plsc.load_gather(cf_v, [a0])
                c1 = plsc.load_gather(cf_v, [a0 + 16])
                rb[r, pl.ds(0, 16)] = rb[r, pl.ds(0, 16)] + c0
                rb[r, pl.ds(16, 16)] = rb[r, pl.ds(16, 16)] + c1
            return carry

        lax.fori_loop(0, SUB // 4, rows, 0, unroll=False)

    def step(j, s, prefetch):
        wait_gather(j, s)
        process(j, s)
        fire_out(j, s)
        if prefetch:
            p = (s - 1) % NBUF
            wait_out(j - 1, p)
            fire_gather(j - 1 + NBUF, p)

    for s in range(NBUF):
        fire_gather(s, s)
    step(0, 0, False)
    for s in range(1, NBUF):
        step(s, s, True)

    def round_body(r, carry):
        j0 = r * NBUF
        for s in range(NBUF):
            step(j0 + s, s, True)
        return carry

    lax.fori_loop(1, NROUND - 1, round_body, 0, unroll=False)

    j0 = (NROUND - 1) * NBUF
    step(j0, 0, True)
    for s in range(1, NBUF):
        step(j0 + s, s, False)
    for s in range(NBUF):
        wait_out(j0 + s, s)


def kernel(cond_table, cat_table, cond_ids, cat_ids, cat_start):
    cond_flat = cond_table.reshape(-1)
    ci = cond_ids.reshape(-1)
    ca = cat_ids.reshape(-1)
    cs = jnp.pad(cat_start, (0, 32 - cat_start.shape[0]))

    mesh = plsc.VectorSubcoreMesh(core_axis_name="c", subcore_axis_name="s")
    f = pl.kernel(
        _body,
        out_type=jax.ShapeDtypeStruct((N_TOT, EMBED), jnp.float32),
        mesh=mesh,
        compiler_params=pltpu.CompilerParams(needs_layout_passes=False,
                                             use_tc_tiling_on_sc=False),
        scratch_types=(
            [
                pltpu.VMEM((32,), jnp.int32),
                pltpu.VMEM((EMBED * (N_COND + 1),), jnp.float32),
                pltpu.VMEM((CHUNK,), jnp.int32),
                pltpu.VMEM((CHUNK,), jnp.int32),
                pltpu.VMEM((NSUB, SUB), jnp.int32),
                pltpu.VMEM((CHUNK,), jnp.int32),
            ]
            + [pltpu.VMEM((SUB, EMBED), jnp.float32) for _ in range(NBUF)]
            + [pltpu.SemaphoreType.DMA for _ in range(2 * NBUF)]
        ),
    )
    out = f(cond_flat, cat_table, ci, ca, cs)
    return out.reshape(B, N_COND, EMBED)


# speedup vs baseline: 2.9891x; 1.0047x over previous
import jax
import jax.numpy as jnp
from jax import lax
from jax.experimental import pallas as pl
from jax.experimental.pallas import tpu as pltpu
from jax.experimental.pallas import tpu_sc as plsc

N_COND = 26
N_CAT = 38461
EMBED = 32
B = 16384

N_TOT = B * N_COND
NW = 32
CHUNK = N_TOT // NW
SUB = 128
NSUB = CHUNK // SUB
GROUPS = SUB // 16
NBUF = 4
NROUND = NSUB // NBUF


def _body(cond_table, cat_table, cond_ids, cat_ids, cat_start, out,
          cs_v, cf_v, ci_v, ca_v, idx2, cv_v, rb0, rb1, rb2, rb3,
          g0, g1, g2, g3, o0, o1, o2, o3):
    rbs = [rb0, rb1, rb2, rb3]
    gsems = [g0, g1, g2, g3]
    osems = [o0, o1, o2, o3]

    wid = lax.axis_index("s") * 2 + lax.axis_index("c")

    pltpu.sync_copy(cat_start, cs_v)
    pltpu.sync_copy(cond_table, cf_v)
    pltpu.sync_copy(cond_ids.at[pl.ds(wid * NSUB, NSUB)], ci_v)
    pltpu.sync_copy(cat_ids.at[pl.ds(wid * NSUB, NSUB)], ca_v)
    base = wid * CHUNK

    def idx_pass(j, carry):
        for k in range(GROUPS):
            c16 = ci_v[j, pl.ds(k * 16, 16)]
            a16 = ca_v[j, pl.ds(k * 16, 16)]
            s16 = plsc.load_gather(cs_v, [c16])
            idx2[j, pl.ds(k * 16, 16)] = s16 + a16
        return carry

    lax.fori_loop(0, NSUB, idx_pass, 0, unroll=False)

    iota = lax.iota(jnp.int32, 16)

    def fire_gather(j, s):
        pltpu.async_copy(cat_table.at[idx2.at[j]], rbs[s], gsems[s])

    def wait_gather(j, s):
        pltpu.make_async_copy(cat_table.at[idx2.at[j]], rbs[s],
                              gsems[s]).wait()

    def fire_out(j, s):
        pltpu.async_copy(rbs[s], out.at[pl.ds(base + j * SUB, SUB)],
                         osems[s])

    def wait_out(j, s):
        pltpu.make_async_copy(rbs[s], out.at[pl.ds(base + j * SUB, SUB)],
                              osems[s]).wait()

    def process(j, s):
        rb = rbs[s]
        j16 = jnp.full((16,), j, jnp.int32)

        def rows(r0, carry):
            for u in range(4):
                r = r0 * 4 + u
                cid16 = plsc.load_gather(ci_v, [j16, jnp.full((16,), r, jnp.int32)])
                c0 = plsc.load_gather(cf_v, [cid16, iota])
                c1 = plsc.load_gather(cf_v, [cid16, iota + 16])
                rb[r, pl.ds(0, 16)] = rb[r, pl.ds(0, 16)] + c0
                rb[r, pl.ds(16, 16)] = rb[r, pl.ds(16, 16)] + c1
            return carry

        lax.fori_loop(0, SUB // 4, rows, 0, unroll=False)

    def step(j, s, prefetch):
        wait_gather(j, s)
        process(j, s)
        fire_out(j, s)
        if prefetch:
            p = (s - 1) % NBUF
            wait_out(j - 1, p)
            fire_gather(j - 1 + NBUF, p)

    for s in range(NBUF):
        fire_gather(s, s)
    step(0, 0, False)
    for s in range(1, NBUF):
        step(s, s, True)

    def round_body(r, carry):
        j0 = r * NBUF
        for s in range(NBUF):
            step(j0 + s, s, True)
        return carry

    lax.fori_loop(1, NROUND - 1, round_body, 0, unroll=False)

    j0 = (NROUND - 1) * NBUF
    step(j0, 0, True)
    for s in range(1, NBUF):
        step(j0 + s, s, False)
    for s in range(NBUF):
        wait_out(j0 + s, s)


def kernel(cond_table, cat_table, cond_ids, cat_ids, cat_start):
    cs = jnp.pad(cat_start, (0, 32 - cat_start.shape[0]))
    ci = cond_ids.reshape(N_TOT // SUB, SUB)
    ca = cat_ids.reshape(N_TOT // SUB, SUB)

    mesh = plsc.VectorSubcoreMesh(core_axis_name="c", subcore_axis_name="s")
    f = pl.kernel(
        _body,
        out_type=jax.ShapeDtypeStruct((N_TOT, EMBED), jnp.float32),
        mesh=mesh,
        compiler_params=pltpu.CompilerParams(needs_layout_passes=False,
                                             use_tc_tiling_on_sc=False),
        scratch_types=(
            [
                pltpu.VMEM((32,), jnp.int32),
                pltpu.VMEM((N_COND + 1, EMBED), jnp.float32),
                pltpu.VMEM((NSUB, SUB), jnp.int32),
                pltpu.VMEM((NSUB, SUB), jnp.int32),
                pltpu.VMEM((NSUB, SUB), jnp.int32),
                pltpu.VMEM((NSUB, SUB), jnp.int32),
            ]
            + [pltpu.VMEM((SUB, EMBED), jnp.float32) for _ in range(NBUF)]
            + [pltpu.SemaphoreType.DMA for _ in range(2 * NBUF)]
        ),
    )
    out = f(cond_table, cat_table, ci, ca, cs)
    return out.reshape(B, N_COND, EMBED)


# speedup vs baseline: 3.8121x; 1.2753x over previous
import jax
import jax.numpy as jnp
from jax import lax
from jax.experimental import pallas as pl
from jax.experimental.pallas import tpu as pltpu
from jax.experimental.pallas import tpu_sc as plsc

N_COND = 26
N_CAT = 38461
EMBED = 32
B = 16384

N_TOT = B * N_COND
NW = 32
BPW = B // NW
CHUNK = N_TOT // NW
SUB = 128
NSUB = CHUNK // SUB
NBUF = 4
NROUND = NSUB // NBUF
PITCH = 133


def _body(cond_flat, cat_table, cond_ids, cat_ids, cat_start, out,
          cs_v, cf_v, ci_v, ca_v, idx2, st0, st1, rb0, rb1, rb2, rb3,
          g0, g1, g2, g3, o0, o1):
    rbs = [rb0, rb1, rb2, rb3]
    sts = [st0, st1]
    gsems = [g0, g1, g2, g3]
    osems = [o0, o1]

    wid = lax.axis_index("s") * 2 + lax.axis_index("c")
    base = wid * CHUNK
    wb = wid * BPW

    pltpu.sync_copy(cat_start, cs_v)
    pltpu.sync_copy(cond_flat, cf_v)
    pltpu.sync_copy(cond_ids.at[pl.ds(base, CHUNK)], ci_v)
    pltpu.sync_copy(cat_ids.at[pl.ds(base, CHUNK)], ca_v)

    iota = lax.iota(jnp.int32, 16)

    def idx_c(c, carry):
        def idx_g(g, fp16):
            cid = plsc.load_gather(ci_v, [fp16])
            cat = plsc.load_gather(ca_v, [fp16])
            s16 = plsc.load_gather(cs_v, [cid])
            u = c * 4 + lax.div(g, 8)
            k = lax.rem(g, 8)
            idx2[u, pl.ds(k * 16, 16)] = s16 + cat
            return fp16 + 16 * N_COND

        lax.fori_loop(0, BPW // 16, idx_g, iota * N_COND + c, unroll=False)
        return carry

    lax.fori_loop(0, N_COND, idx_c, 0, unroll=False)

    def fire_gather(u, s):
        pltpu.async_copy(cat_table.at[idx2.at[u]], rbs[s], gsems[s])

    def wait_gather(u, s):
        pltpu.make_async_copy(cat_table.at[idx2.at[u]], rbs[s],
                              gsems[s]).wait()

    def out_dst(u):
        c = lax.div(u, 4)
        blk = lax.rem(u, 4)
        return out.at[pl.ds(c, 1), :, pl.ds(wb // 128 + blk, 1), :, :]

    def fire_out(u, stg):
        pltpu.async_copy(sts[stg].at[:, :, :, :, pl.ds(0, 128)],
                         out_dst(u), osems[stg])

    def wait_out(u, stg):
        pltpu.make_async_copy(sts[stg].at[:, :, :, :, pl.ds(0, 128)],
                              out_dst(u), osems[stg]).wait()

    zero16 = jnp.zeros((16,), jnp.int32)
    e1_lo = lax.shift_right_logical(iota, 3)
    e1_hi = e1_lo + 2
    e2_16 = lax.bitwise_and(iota, 7)

    def process(u, s, stg):
        rb = rbs[s]
        st = sts[stg]
        blk = lax.rem(u, 4)
        c = lax.div(u, 4)
        fp0 = (blk * 128) * N_COND + c

        def rows(r0, carry):
            for v in range(2):
                r = r0 * 2 + v
                fp16 = jnp.full((16,), fp0 + r * N_COND, jnp.int32)
                cid16 = plsc.load_gather(ci_v, [fp16])
                cb16 = cid16 * EMBED
                c0 = plsc.load_gather(cf_v, [cb16 + iota])
                c1 = plsc.load_gather(cf_v, [cb16 + (iota + 16)])
                r16 = jnp.full((16,), r, jnp.int32)
                plsc.store_scatter(st, [zero16, e1_lo, zero16, e2_16, r16],
                                   rb[r, pl.ds(0, 16)] + c0)
                plsc.store_scatter(st, [zero16, e1_hi, zero16, e2_16, r16],
                                   rb[r, pl.ds(16, 16)] + c1)
            return carry

        lax.fori_loop(0, SUB // 2, rows, 0, unroll=False)

    def step(u, s, stg, st_wait, prefetch):
        wait_gather(u, s)
        if st_wait:
            wait_out(u - 2, stg)
        process(u, s, stg)
        fire_out(u, stg)
        if prefetch:
            fire_gather(u + NBUF, s)

    for s in range(NBUF):
        fire_gather(s, s)
    step(0, 0, 0, False, True)
    step(1, 1, 1, False, True)
    step(2, 2, 0, True, True)
    step(3, 3, 1, True, True)

    def round_body(rnd, carry):
        u0 = rnd * NBUF
        for s in range(NBUF):
            step(u0 + s, s, s % 2, True, True)
        return carry

    lax.fori_loop(1, NROUND - 1, round_body, 0, unroll=False)

    u0 = (NROUND - 1) * NBUF
    for s in range(NBUF):
        step(u0 + s, s, s % 2, True, False)
    wait_out(NSUB - 2, 0)
    wait_out(NSUB - 1, 1)


def kernel(cond_table, cat_table, cond_ids, cat_ids, cat_start):
    cs = jnp.pad(cat_start, (0, 32 - cat_start.shape[0]))
    cond_flat = cond_table.reshape(-1)
    ci = cond_ids.reshape(-1)
    ca = cat_ids.reshape(-1)

    mesh = plsc.VectorSubcoreMesh(core_axis_name="c", subcore_axis_name="s")
    f = pl.kernel(
        _body,
        out_type=jax.ShapeDtypeStruct((N_COND, 4, B // 128, 8, 128),
                                      jnp.float32),
        mesh=mesh,
        compiler_params=pltpu.CompilerParams(needs_layout_passes=False,
                                             use_tc_tiling_on_sc=False),
        scratch_types=(
            [
                pltpu.VMEM((32,), jnp.int32),
                pltpu.VMEM((EMBED * (N_COND + 1),), jnp.float32),
                pltpu.VMEM((CHUNK,), jnp.int32),
                pltpu.VMEM((CHUNK,), jnp.int32),
                pltpu.VMEM((NSUB, SUB), jnp.int32),
            ]
            + [pltpu.VMEM((1, 4, 1, 8, PITCH), jnp.float32)
               for _ in range(2)]
            + [pltpu.VMEM((SUB, EMBED), jnp.float32) for _ in range(NBUF)]
            + [pltpu.SemaphoreType.DMA for _ in range(NBUF + 2)]
        ),
    )
    out5 = f(cond_flat, cat_table, ci, ca, cs)
    return out5.transpose(2, 4, 0, 1, 3).reshape(B, N_COND, EMBED)
